# baseline (speedup 1.0000x reference)
import functools

import jax
import jax.numpy as jnp
from jax.experimental import pallas as pl
from jax.experimental.pallas import tpu as pltpu

_B = 1024
_K = 8
_N = 8192
_BJ = 2048


def _mm_kernel(wt_ref, b_ref, ct_ref, o_ref):
    wb = jnp.concatenate([wt_ref[...], b_ref[...]], axis=0)
    ct = jnp.concatenate(
        [ct_ref[...], jnp.ones((1, _B), jnp.float32)], axis=0)
    o_ref[...] = jax.lax.dot_general(wb, ct, (((0,), (0,)), ((), ())),
                                     preferred_element_type=jnp.float32)


@functools.partial(jax.jit, static_argnames=())
def kernel(condition, W, b):
    yt = pl.pallas_call(
        _mm_kernel,
        grid=(_N // _BJ,),
        in_specs=[
            pl.BlockSpec((_K, _BJ), lambda i: (0, i)),
            pl.BlockSpec((1, _BJ), lambda i: (0, i)),
            pl.BlockSpec((_K, _B), lambda i: (0, 0)),
        ],
        out_specs=pl.BlockSpec((_BJ, _B), lambda i: (i, 0)),
        out_shape=jax.ShapeDtypeStruct((_N, _B), jnp.float32),
    )(W.T, b.reshape(1, _N), condition.T)
    return yt.reshape(32, 16, 16, _B).transpose(3, 0, 1, 2)


# speedup vs baseline: 1.0149x; 1.0149x over previous
import functools

import jax
import jax.numpy as jnp
from jax.experimental import pallas as pl
from jax.experimental.pallas import tpu as pltpu

_B = 1024
_K = 8
_N = 8192
_BJ = 1024


def _mm_kernel(wt_ref, b_ref, ct_ref, o_ref):
    wb = jnp.concatenate([wt_ref[...], b_ref[...]], axis=0)
    ct = jnp.concatenate(
        [ct_ref[...], jnp.ones((1, _B), jnp.float32)], axis=0)
    o_ref[...] = jax.lax.dot_general(
        wb.astype(jnp.bfloat16), ct.astype(jnp.bfloat16),
        (((0,), (0,)), ((), ())),
        preferred_element_type=jnp.float32)


@functools.partial(jax.jit, static_argnames=())
def kernel(condition, W, b):
    yt = pl.pallas_call(
        _mm_kernel,
        grid=(_N // _BJ,),
        in_specs=[
            pl.BlockSpec((_K, _BJ), lambda i: (0, i)),
            pl.BlockSpec((1, _BJ), lambda i: (0, i)),
            pl.BlockSpec((_K, _B), lambda i: (0, 0)),
        ],
        out_specs=pl.BlockSpec((_BJ, _B), lambda i: (i, 0)),
        out_shape=jax.ShapeDtypeStruct((_N, _B), jnp.float32),
    )(W.T, b.reshape(1, _N), condition.T)
    return yt.reshape(32, 16, 16, _B).transpose(3, 0, 1, 2)
